# baseline (speedup 1.0000x reference)
import jax
import jax.numpy as jnp
from jax import lax
from jax.experimental import pallas as pl
from jax.experimental.pallas import tpu as pltpu

_HB = 13


def _make_body(b_dim, h_dim, w_dim, a_dim, c_dim):
    hsteps = h_dim // _HB
    steps = b_dim * hsteps

    def body(p_hbm, g_hbm, out_ref, pbuf, gbuf, sem):
        i = pl.program_id(0)
        slot = lax.rem(i, 2)
        nxt = lax.rem(i + 1, 2)

        def start(step, slot_):
            b = step // hsteps
            h0 = lax.rem(step, hsteps) * _HB
            for a in range(a_dim):
                pltpu.make_async_copy(
                    p_hbm.at[b, pl.ds(h0, _HB), :, a, :],
                    pbuf.at[slot_, a],
                    sem.at[slot_, 0, a],
                ).start()
                pltpu.make_async_copy(
                    g_hbm.at[b, pl.ds(h0, _HB), :, a, :],
                    gbuf.at[slot_, a],
                    sem.at[slot_, 1, a],
                ).start()

        @pl.when(i == 0)
        def _prologue():
            start(i, slot)

        @pl.when(i + 1 < steps)
        def _prefetch():
            start(i + 1, nxt)

        s = jnp.float32(0.0)
        for a in range(a_dim):
            pltpu.make_async_copy(
                p_hbm.at[0, pl.ds(0, _HB), :, a, :], pbuf.at[slot, a],
                sem.at[slot, 0, a]).wait()
            pltpu.make_async_copy(
                g_hbm.at[0, pl.ds(0, _HB), :, a, :], gbuf.at[slot, a],
                sem.at[slot, 1, a]).wait()
            p = pbuf[slot, a]
            g = gbuf[slot, a]
            c = lax.broadcasted_iota(jnp.int32, p.shape, 2)
            g4 = g[..., 4:5]
            mse_w = jnp.where(c < 4, 5.0 * g4, 0.0)
            bce_w = jnp.where(c == 4, 0.5 + 0.5 * g4,
                              jnp.where(c >= 5, g4, 0.0))
            d = g - p
            bce = jnp.maximum(p, 0.0) - p * g + jnp.log1p(jnp.exp(-jnp.abs(p)))
            s = s + jnp.sum(mse_w * (d * d) + bce_w * bce)

        @pl.when(i == 0)
        def _init():
            out_ref[0, 0] = s

        @pl.when(i != 0)
        def _acc():
            out_ref[0, 0] = out_ref[0, 0] + s

    return body, steps


def kernel(pred, gt):
    b_dim, h_dim, w_dim, a_dim, c_dim = pred.shape
    body, steps = _make_body(b_dim, h_dim, w_dim, a_dim, c_dim)
    out = pl.pallas_call(
        body,
        grid=(steps,),
        in_specs=[
            pl.BlockSpec(memory_space=pl.ANY),
            pl.BlockSpec(memory_space=pl.ANY),
        ],
        out_specs=pl.BlockSpec((1, 1), lambda i: (0, 0),
                               memory_space=pltpu.SMEM),
        out_shape=jax.ShapeDtypeStruct((1, 1), jnp.float32),
        scratch_shapes=[
            pltpu.VMEM((2, a_dim, _HB, w_dim, c_dim), jnp.float32),
            pltpu.VMEM((2, a_dim, _HB, w_dim, c_dim), jnp.float32),
            pltpu.SemaphoreType.DMA((2, 2, a_dim)),
        ],
    )(pred, gt)
    return out[0, 0] * (1.0 / b_dim)


# speedup vs baseline: 1.0807x; 1.0807x over previous
import jax
import jax.numpy as jnp
from jax import lax
from jax.experimental import pallas as pl
from jax.experimental.pallas import tpu as pltpu

_HB = 13


def _make_body(b_dim, h_dim, w_dim, a_dim, c_dim):
    hsteps = h_dim // _HB
    steps = b_dim * hsteps

    def body(p_hbm, g_hbm, out_ref, pbuf, gbuf, sem):
        i = pl.program_id(0)
        slot = lax.rem(i, 3)
        nxt = lax.rem(i + 2, 3)

        def start(step, slot_):
            b = step // hsteps
            h0 = lax.rem(step, hsteps) * _HB
            for a in range(a_dim):
                pltpu.make_async_copy(
                    p_hbm.at[b, pl.ds(h0, _HB), :, a, :],
                    pbuf.at[slot_, a],
                    sem.at[slot_, 0, a],
                ).start()
                pltpu.make_async_copy(
                    g_hbm.at[b, pl.ds(h0, _HB), :, a, :],
                    gbuf.at[slot_, a],
                    sem.at[slot_, 1, a],
                ).start()

        @pl.when(i == 0)
        def _prologue():
            start(0, 0)
            start(1, 1)

        @pl.when(i + 2 < steps)
        def _prefetch():
            start(i + 2, nxt)

        for a in range(a_dim):
            pltpu.make_async_copy(
                p_hbm.at[0, pl.ds(0, _HB), :, a, :], pbuf.at[slot, a],
                sem.at[slot, 0, a]).wait()
            pltpu.make_async_copy(
                g_hbm.at[0, pl.ds(0, _HB), :, a, :], gbuf.at[slot, a],
                sem.at[slot, 1, a]).wait()

        s = jnp.float32(0.0)
        for a in range(a_dim):
            p = pbuf[slot, a]
            g = gbuf[slot, a]
            c = lax.broadcasted_iota(jnp.int32, p.shape, 2)
            g4 = g[..., 4:5]
            mse_w = jnp.where(c < 4, 5.0 * g4, 0.0)
            bce_w = jnp.where(c == 4, 0.5 + 0.5 * g4,
                              jnp.where(c >= 5, g4, 0.0))
            d = g - p
            bce = jnp.maximum(p, 0.0) - p * g + jnp.log1p(jnp.exp(-jnp.abs(p)))
            s = s + jnp.sum(mse_w * (d * d) + bce_w * bce)

        @pl.when(i == 0)
        def _init():
            out_ref[0, 0] = s

        @pl.when(i != 0)
        def _acc():
            out_ref[0, 0] = out_ref[0, 0] + s

    return body, steps


def kernel(pred, gt):
    b_dim, h_dim, w_dim, a_dim, c_dim = pred.shape
    body, steps = _make_body(b_dim, h_dim, w_dim, a_dim, c_dim)
    out = pl.pallas_call(
        body,
        grid=(steps,),
        in_specs=[
            pl.BlockSpec(memory_space=pl.ANY),
            pl.BlockSpec(memory_space=pl.ANY),
        ],
        out_specs=pl.BlockSpec((1, 1), lambda i: (0, 0),
                               memory_space=pltpu.SMEM),
        out_shape=jax.ShapeDtypeStruct((1, 1), jnp.float32),
        scratch_shapes=[
            pltpu.VMEM((3, a_dim, _HB, w_dim, c_dim), jnp.float32),
            pltpu.VMEM((3, a_dim, _HB, w_dim, c_dim), jnp.float32),
            pltpu.SemaphoreType.DMA((3, 2, a_dim)),
        ],
    )(pred, gt)
    return out[0, 0] * (1.0 / b_dim)
